# Initial kernel scaffold; baseline (speedup 1.0000x reference)
import jax
import jax.numpy as jnp
from jax.experimental import pallas as pl


def kernel(att_ws):
    raise NotImplementedError("write your pallas kernel here")


# speedup vs baseline: 1.5254x; 1.5254x over previous
import jax
import jax.numpy as jnp
from jax.experimental import pallas as pl

_T_OUT = 8192
_T_IN = 4096
_BR = 256


def _body(x_ref, out_ref):
    i = pl.program_id(0)
    x = x_ref[...]
    rowmax = jnp.max(x, axis=1, keepdims=True)
    col = jax.lax.broadcasted_iota(jnp.int32, x.shape, 1)
    first = jnp.min(jnp.where(x == rowmax, col, _T_IN), axis=1, keepdims=True)
    contrib = jnp.sum((col == first).astype(jnp.int32), axis=0, keepdims=True)

    @pl.when(i == 0)
    def _():
        out_ref[...] = jnp.zeros_like(out_ref)

    out_ref[...] += contrib


def kernel(att_ws):
    counts = pl.pallas_call(
        _body,
        grid=(_T_OUT // _BR,),
        in_specs=[pl.BlockSpec((_BR, _T_IN), lambda i: (i, 0))],
        out_specs=pl.BlockSpec((1, _T_IN), lambda i: (0, 0)),
        out_shape=jax.ShapeDtypeStruct((1, _T_IN), jnp.int32),
    )(att_ws)
    return counts.reshape(-1)
